# baseline (speedup 1.0000x reference)
import functools

import jax
import jax.numpy as jnp
from jax import lax
from jax.experimental import pallas as pl
from jax.experimental.pallas import tpu as pltpu
from jax.experimental.pallas import tpu_sc as plsc

_D = 768
_SEQ = 77
_BATCH = 1024
_NC = 2
_NS = 16
_NW = _NC * _NS
_BPW = _BATCH // _NW
_LANES = 16
_PPW = _D // (2 * _LANES)
_WS = 2
_W = _WS * _BPW
_NWIN = _SEQ // _WS


def _sc_embed(ids_w, tok_w, pos_pk):
  mesh = plsc.VectorSubcoreMesh(core_axis_name="c", subcore_axis_name="s")

  @functools.partial(
      pl.kernel,
      mesh=mesh,
      out_type=jax.ShapeDtypeStruct((_SEQ, _BATCH, _D), jnp.float32),
      scratch_types=[
          pltpu.VMEM((_SEQ * _BPW,), jnp.int32),
          pltpu.VMEM((_SEQ * _PPW * _LANES,), jnp.int32),
          pltpu.VMEM((_W, _D), jnp.float32),
          pltpu.VMEM((_W, _D), jnp.float32),
          pltpu.SemaphoreType.DMA,
          pltpu.SemaphoreType.DMA,
          pltpu.SemaphoreType.DMA,
          pltpu.SemaphoreType.DMA,
      ],
  )
  def k(ids_hbm, tab_hbm, pos_hbm, out_hbm, idx_v, pos_v, buf_a, buf_b,
        gsem_a, gsem_b, wsem_a, wsem_b):
    wid = lax.axis_index("s") * _NC + lax.axis_index("c")
    b0 = wid * _BPW
    pltpu.sync_copy(ids_hbm.at[pl.ds(wid * _SEQ * _BPW, _SEQ * _BPW)], idx_v)
    pltpu.sync_copy(pos_hbm, pos_v)

    def g_start(kw, buf, gsem):
      pltpu.async_copy(tab_hbm.at[idx_v.at[pl.ds(kw * _W, _W)]], buf, gsem)

    def g_wait(kw, buf, gsem):
      pltpu.make_async_copy(
          tab_hbm.at[idx_v.at[pl.ds(kw * _W, _W)]], buf, gsem).wait()

    def w_start(kw, buf, wsem):
      pltpu.async_copy(
          buf.reshape(_WS, _BPW, _D),
          out_hbm.at[pl.ds(kw * _WS, _WS), pl.ds(b0, _BPW)], wsem)

    def w_wait(kw, buf, wsem):
      pltpu.make_async_copy(
          buf.reshape(_WS, _BPW, _D),
          out_hbm.at[pl.ds(kw * _WS, _WS), pl.ds(b0, _BPW)], wsem).wait()

    def add_pos(s, buf, r0):
      pchunks = []
      for cp in range(_PPW):
        packed = pos_v[pl.ds(s * (_D // 2) + _LANES * cp, _LANES)]
        pchunks.append(lax.bitcast_convert_type(packed << 16, jnp.float32))
        pchunks.append(
            lax.bitcast_convert_type(packed & jnp.int32(-65536), jnp.float32))

      @plsc.parallel_loop(r0, r0 + _BPW)
      def _(i):
        for c in range(_D // _LANES):
          sl = pl.ds(c * _LANES, _LANES)
          buf[i, sl] = buf[i, sl] + pchunks[c]

    def add_win(kw, buf):
      add_pos(kw * _WS, buf, 0)
      add_pos(kw * _WS + 1, buf, _BPW)

    g_start(0, buf_a, gsem_a)
    g_start(1, buf_b, gsem_b)

    npair = _NWIN // 2

    def pair_body(t, carry):
      kw = 2 * t
      g_wait(kw, buf_a, gsem_a)
      add_win(kw, buf_a)
      w_start(kw, buf_a, wsem_a)
      g_wait(kw + 1, buf_b, gsem_b)
      add_win(kw + 1, buf_b)
      w_start(kw + 1, buf_b, wsem_b)

      @pl.when(kw + 2 < _NWIN)
      def _():
        w_wait(kw, buf_a, wsem_a)
        g_start(kw + 2, buf_a, gsem_a)

      @pl.when(kw + 3 < _NWIN)
      def _():
        w_wait(kw + 1, buf_b, wsem_b)
        g_start(kw + 3, buf_b, gsem_b)

      return carry

    lax.fori_loop(0, npair, pair_body, 0)

    s_last = _SEQ - 1
    w_wait(_NWIN - 2, buf_a, wsem_a)
    pltpu.async_copy(
        tab_hbm.at[idx_v.at[pl.ds(s_last * _BPW, _BPW)]],
        buf_a.at[pl.ds(0, _BPW)], gsem_a)
    pltpu.make_async_copy(
        tab_hbm.at[idx_v.at[pl.ds(s_last * _BPW, _BPW)]],
        buf_a.at[pl.ds(0, _BPW)], gsem_a).wait()
    add_pos(s_last, buf_a, 0)
    pltpu.async_copy(
        buf_a.at[pl.ds(0, _BPW)], out_hbm.at[s_last, pl.ds(b0, _BPW)], wsem_a)
    pltpu.make_async_copy(
        buf_a.at[pl.ds(0, _BPW)], out_hbm.at[s_last, pl.ds(b0, _BPW)],
        wsem_a).wait()
    w_wait(_NWIN - 1, buf_b, wsem_b)

  return k(ids_w, tok_w, pos_pk)


def kernel(input_ids, token_embedding_weight, position_embedding_weight):
  ids = input_ids.astype(jnp.int32)
  ids_w = ids.T.reshape(_SEQ, _NW, _BPW).transpose(1, 0, 2).reshape(-1)
  bits = lax.bitcast_convert_type(
      position_embedding_weight.astype(jnp.bfloat16), jnp.uint16
  ).reshape(-1, 2, _LANES).astype(jnp.uint32)
  pos_pk = lax.bitcast_convert_type(
      bits[:, 0, :] | (bits[:, 1, :] << 16), jnp.int32).reshape(-1)
  out_t = _sc_embed(ids_w, token_embedding_weight, pos_pk)
  return out_t.transpose(1, 0, 2)


# speedup vs baseline: 1.0359x; 1.0359x over previous
import functools

import jax
import jax.numpy as jnp
from jax import lax
from jax.experimental import pallas as pl
from jax.experimental.pallas import tpu as pltpu
from jax.experimental.pallas import tpu_sc as plsc

_D = 768
_SEQ = 77
_BATCH = 1024
_NC = 2
_NS = 16
_NW = _NC * _NS
_BPW = _BATCH // _NW
_LANES = 16
_PPW = _D // (2 * _LANES)


def _sc_embed(ids_w, tok_w, pos_pk):
  mesh = plsc.VectorSubcoreMesh(core_axis_name="c", subcore_axis_name="s")

  @functools.partial(
      pl.kernel,
      mesh=mesh,
      out_type=jax.ShapeDtypeStruct((_SEQ, _BATCH, _D), jnp.float32),
      scratch_types=[
          pltpu.VMEM((_SEQ * _BPW,), jnp.int32),
          pltpu.VMEM((_SEQ, _D), jnp.float32),
          pltpu.VMEM((_BPW, _D), jnp.float32),
          pltpu.VMEM((_BPW, _D), jnp.float32),
          pltpu.SemaphoreType.DMA,
          pltpu.SemaphoreType.DMA,
          pltpu.SemaphoreType.DMA,
          pltpu.SemaphoreType.DMA,
          pltpu.SemaphoreType.DMA,
      ],
  )
  def k(ids_hbm, tab_hbm, pos_hbm, out_hbm, idx_v, pos_v, buf_a, buf_b,
        gsem_a, gsem_b, wsem_a, wsem_b, psem):
    wid = lax.axis_index("s") * _NC + lax.axis_index("c")
    b0 = wid * _BPW
    pltpu.sync_copy(ids_hbm.at[pl.ds(wid * _SEQ * _BPW, _SEQ * _BPW)], idx_v)
    pos_copy = pltpu.async_copy(pos_hbm, pos_v, psem)

    def g_start(s, buf, gsem):
      pltpu.async_copy(tab_hbm.at[idx_v.at[pl.ds(s * _BPW, _BPW)]], buf, gsem)

    def g_wait(s, buf, gsem):
      pltpu.make_async_copy(
          tab_hbm.at[idx_v.at[pl.ds(s * _BPW, _BPW)]], buf, gsem).wait()

    def w_start(s, buf, wsem):
      pltpu.async_copy(buf, out_hbm.at[s, pl.ds(b0, _BPW)], wsem)

    def w_wait(s, buf, wsem):
      pltpu.make_async_copy(buf, out_hbm.at[s, pl.ds(b0, _BPW)], wsem).wait()

    def add_pos(s, buf):
      pchunks = [
          pos_v[s, pl.ds(c * _LANES, _LANES)]
          for c in range(_D // _LANES)
      ]

      @plsc.parallel_loop(0, _BPW)
      def _(i):
        for c in range(_D // _LANES):
          sl = pl.ds(c * _LANES, _LANES)
          buf[i, sl] = buf[i, sl] + pchunks[c]

    g_start(0, buf_a, gsem_a)
    g_start(1, buf_b, gsem_b)
    pos_copy.wait()

    npair = _SEQ // 2

    def pair_body(t, carry):
      s = 2 * t
      g_wait(s, buf_a, gsem_a)
      add_pos(s, buf_a)
      w_start(s, buf_a, wsem_a)
      g_wait(s + 1, buf_b, gsem_b)
      add_pos(s + 1, buf_b)
      w_start(s + 1, buf_b, wsem_b)

      @pl.when(s + 2 < _SEQ)
      def _():
        w_wait(s, buf_a, wsem_a)
        g_start(s + 2, buf_a, gsem_a)

      @pl.when(s + 3 < _SEQ)
      def _():
        w_wait(s + 1, buf_b, wsem_b)
        g_start(s + 3, buf_b, gsem_b)

      return carry

    lax.fori_loop(0, npair, pair_body, 0)

    s_last = _SEQ - 1
    g_wait(s_last, buf_a, gsem_a)
    add_pos(s_last, buf_a)
    w_start(s_last, buf_a, wsem_a)
    w_wait(s_last, buf_a, wsem_a)
    w_wait(s_last - 1, buf_b, wsem_b)

  return k(ids_w, tok_w, pos_pk)


def kernel(input_ids, token_embedding_weight, position_embedding_weight):
  ids = input_ids.astype(jnp.int32)
  ids_w = ids.reshape(_NW, _BPW, _SEQ).transpose(0, 2, 1).reshape(-1)
  out_t = _sc_embed(ids_w, token_embedding_weight,
                    position_embedding_weight)
  return out_t.transpose(1, 0, 2)
